# Initial kernel scaffold; baseline (speedup 1.0000x reference)
import jax
import jax.numpy as jnp
from jax.experimental import pallas as pl


def kernel(f, pos, batch, Wq, Wk1, Wk2, Wv1, Wv2, Wdot):
    raise NotImplementedError("write your pallas kernel here")


# speedup vs baseline: 2.7322x; 2.7322x over previous
import functools

import jax
import jax.numpy as jnp
from jax import lax
from jax.experimental import pallas as pl
from jax.experimental.pallas import tpu as pltpu
from jax.experimental.pallas import tpu_sc as plsc

_N = 10000
_DIN = 32
_DOUT = 32
_DK = 16
_DQ = 16
_NB = 10
_HID = 16
_R = 0.22
_K = 16

_B1 = 200
_B3 = 200
_EB = _B3 * _K
_E = _N * _K

_NW = 32
_PERW = _E // _NW
_CH = 1000


def _edges_body(pos_ref, post_ref, src_ref, d2_ref):
    pb = pos_ref[...]
    pt = post_ref[...]
    x2 = jnp.sum(pt * pt, axis=0, keepdims=True)
    p2 = jnp.sum(pb * pb, axis=1, keepdims=True)
    cross = jnp.dot(pb, pt, preferred_element_type=jnp.float32)
    d2 = jnp.maximum(p2 + x2 - 2.0 * cross, 0.0)
    base = pl.program_id(0) * _B1
    col = lax.broadcasted_iota(jnp.int32, (_B1, _N), 1)
    row = lax.broadcasted_iota(jnp.int32, (_B1, _N), 0) + base
    ok = (col != row) & (d2 < _R * _R)
    big = jnp.float32(1e30)
    score = jnp.where(ok, d2, big)
    srcs = jnp.zeros((_B1, _K), jnp.int32)
    d2s = jnp.full((_B1, _K), big, jnp.float32)
    kl = lax.broadcasted_iota(jnp.int32, (_B1, _K), 1)
    for k in range(_K):
        m = jnp.min(score, axis=1, keepdims=True)
        idx = jnp.min(jnp.where(score == m, col, _N), axis=1, keepdims=True)
        srcs = jnp.where(kl == k, idx, srcs)
        d2s = jnp.where(kl == k, m, d2s)
        score = jnp.where(col == idx, big, score)
    src_ref[...] = srcs
    d2_ref[...] = d2s


def _build_edges(pos):
    pos_t = pos.T
    return pl.pallas_call(
        _edges_body,
        grid=(_N // _B1,),
        in_specs=[
            pl.BlockSpec((_B1, 3), lambda i: (i, 0)),
            pl.BlockSpec((3, _N), lambda i: (0, 0)),
        ],
        out_specs=[
            pl.BlockSpec((_B1, _K), lambda i: (i, 0)),
            pl.BlockSpec((_B1, _K), lambda i: (i, 0)),
        ],
        out_shape=[
            jax.ShapeDtypeStruct((_N, _K), jnp.int32),
            jax.ShapeDtypeStruct((_N, _K), jnp.float32),
        ],
    )(pos, pos_t)


def _sc_gather(f, posp, src_flat, dst_flat):
    mesh = plsc.VectorSubcoreMesh(core_axis_name="c", subcore_axis_name="s")

    @functools.partial(
        pl.kernel,
        mesh=mesh,
        compiler_params=pltpu.CompilerParams(use_tc_tiling_on_sc=False),
        out_type=[
            jax.ShapeDtypeStruct((_E, _DIN), jnp.float32),
            jax.ShapeDtypeStruct((_E, 16), jnp.float32),
            jax.ShapeDtypeStruct((_E, 16), jnp.float32),
        ],
        scratch_types=[
            pltpu.VMEM((_CH,), jnp.int32),
            pltpu.VMEM((_CH,), jnp.int32),
            pltpu.VMEM((_CH, _DIN), jnp.float32),
            pltpu.VMEM((_CH, 16), jnp.float32),
            pltpu.VMEM((_CH, 16), jnp.float32),
            pltpu.SemaphoreType.DMA,
        ],
    )
    def k(f_hbm, p_hbm, si_hbm, di_hbm, fs_hbm, ps_hbm, pd_hbm,
          si_v, di_v, rf_v, rp_v, rd_v, sem):
        wid = lax.axis_index("s") * 2 + lax.axis_index("c")
        base = wid * _PERW
        for c in range(_PERW // _CH):
            off = base + c * _CH
            pltpu.sync_copy(si_hbm.at[pl.ds(off, _CH)], si_v)
            pltpu.sync_copy(di_hbm.at[pl.ds(off, _CH)], di_v)
            pltpu.async_copy(f_hbm.at[si_v], rf_v, sem).wait()
            pltpu.async_copy(p_hbm.at[si_v], rp_v, sem).wait()
            pltpu.async_copy(p_hbm.at[di_v], rd_v, sem).wait()
            pltpu.sync_copy(rf_v, fs_hbm.at[pl.ds(off, _CH)])
            pltpu.sync_copy(rp_v, ps_hbm.at[pl.ds(off, _CH)])
            pltpu.sync_copy(rd_v, pd_hbm.at[pl.ds(off, _CH)])

    return k(f, posp, src_flat, dst_flat)


def _core_body(f_ref, fs_ref, ps_ref, pd_ref, d2_ref, wq_ref, wdot_ref,
               wk1_ref, ak_ref, wv1_ref, av_ref, r16_ref, g16_ref, r32_ref,
               g32_ref, o_ref):
    fb = f_ref[...]
    fs = fs_ref[...]
    d2 = d2_ref[...]
    valid = d2 < _R * _R
    validf = valid.astype(jnp.float32)

    dot = lambda a, b: jnp.dot(a, b, preferred_element_type=jnp.float32,
                               precision=jax.lax.Precision.HIGHEST)

    q = dot(fb, wq_ref[...]) * (1.0 / jnp.sqrt(jnp.float32(_DIN)))
    qw = dot(q, wdot_ref[...])

    er = lax.broadcasted_iota(jnp.int32, (_EB, _B3), 0)
    nc = lax.broadcasted_iota(jnp.int32, (_EB, _B3), 1)
    oh_en = (er // _K == nc).astype(jnp.float32)
    ekr = lax.broadcasted_iota(jnp.int32, (_EB, _K), 0)
    ekc = lax.broadcasted_iota(jnp.int32, (_EB, _K), 1)
    skm = (ekc == ekr % _K).astype(jnp.float32)

    dv = ps_ref[:, :3] - pd_ref[:, :3]
    elen_e = jnp.sqrt(jnp.sum(dv * dv, axis=1, keepdims=True) + 1e-12)
    step = _R / (_NB + 1)
    centers = (lax.broadcasted_iota(jnp.int32, (1, _NB), 1).astype(jnp.float32)
               + 1.0) * step
    diff = (elen_e - centers) * (1.0 / step)
    emb = (jnp.exp(-diff * diff) * (1.0 / 1.12)) * (_NB ** 0.5)

    inv_nb = 1.0 / jnp.sqrt(jnp.float32(_NB))
    sqrt2 = jnp.sqrt(jnp.float32(2.0))
    hk = sqrt2 * jax.nn.relu(dot(emb, wk1_ref[...]) * inv_nb)
    hv = sqrt2 * jax.nn.relu(dot(emb, wv1_ref[...]) * inv_nb)

    tp_scale = 1.0 / (jnp.sqrt(jnp.float32(_HID)) * jnp.sqrt(jnp.float32(_DIN)))
    tk = dot(fs, ak_ref[...])
    hk_rep = dot(hk, r16_ref[...])
    km = dot(hk_rep * tk, g16_ref[...]) * tp_scale

    tv = dot(fs, av_ref[...])
    hv_rep = dot(hv, r32_ref[...])
    vm = dot(hv_rep * tv, g32_ref[...]) * tp_scale

    qw_e = dot(oh_en, qw)
    lg_e = jnp.sum(qw_e * km, axis=1, keepdims=True) * (
        1.0 / jnp.sqrt(jnp.float32(_DQ * _DK)))

    nr = lax.broadcasted_iota(jnp.int32, (_B3, _EB), 0)
    ec = lax.broadcasted_iota(jnp.int32, (_B3, _EB), 1)
    oh_ne = (ec // _K == nr).astype(jnp.float32)
    lg = dot(oh_ne, lg_e * skm)
    lg = jnp.where(valid, lg, -1e30)
    mx = jnp.max(lg, axis=1, keepdims=True)
    ex = jnp.where(valid, jnp.exp(lg - mx), 0.0)
    den = jnp.sum(ex, axis=1, keepdims=True)
    alpha = ex / (den + 1e-16)

    xcut = 10.0 * (1.0 - elen_e * (1.0 / _R))
    xp = jnp.where(xcut > 0.0, xcut, 1.0)
    cut_e = jnp.where(xcut > 0.0, jnp.exp(-1.0 / xp), 0.0)

    alpha_e = jnp.sum(dot(oh_en, alpha) * skm, axis=1, keepdims=True)
    valid_e = jnp.sum(dot(oh_en, validf) * skm, axis=1, keepdims=True)
    coef_e = jnp.sqrt(alpha_e * cut_e + 1e-12) * valid_e
    o_ref[...] = dot(oh_ne, coef_e * vm)


def _core(f, fs, ps, pd, d2s, Wq, Wdot, Wk1, Wk2, Wv1, Wv2):
    ak = Wk2.reshape(_HID, _DIN, _DK).transpose(1, 0, 2).reshape(_DIN, _HID * _DK)
    av = Wv2.reshape(_HID, _DIN, _DOUT).transpose(1, 0, 2).reshape(_DIN, _HID * _DOUT)
    r16 = (jnp.arange(_HID * _DK)[None, :] // _DK
           == jnp.arange(_HID)[:, None]).astype(jnp.float32)
    g16 = jnp.tile(jnp.eye(_DK, dtype=jnp.float32), (_HID, 1))
    r32 = (jnp.arange(_HID * _DOUT)[None, :] // _DOUT
           == jnp.arange(_HID)[:, None]).astype(jnp.float32)
    g32 = jnp.tile(jnp.eye(_DOUT, dtype=jnp.float32), (_HID, 1))
    full = lambda shape: pl.BlockSpec(shape, lambda i: tuple(0 for _ in shape))
    return pl.pallas_call(
        _core_body,
        grid=(_N // _B3,),
        in_specs=[
            pl.BlockSpec((_B3, _DIN), lambda i: (i, 0)),
            pl.BlockSpec((_EB, _DIN), lambda i: (i, 0)),
            pl.BlockSpec((_EB, 16), lambda i: (i, 0)),
            pl.BlockSpec((_EB, 16), lambda i: (i, 0)),
            pl.BlockSpec((_B3, _K), lambda i: (i, 0)),
            full((_DIN, _DQ)),
            full((_DQ, _DK)),
            full((_NB, _HID)),
            full((_DIN, _HID * _DK)),
            full((_NB, _HID)),
            full((_DIN, _HID * _DOUT)),
            full((_HID, _HID * _DK)),
            full((_HID * _DK, _DK)),
            full((_HID, _HID * _DOUT)),
            full((_HID * _DOUT, _DOUT)),
        ],
        out_specs=pl.BlockSpec((_B3, _DOUT), lambda i: (i, 0)),
        out_shape=jax.ShapeDtypeStruct((_N, _DOUT), jnp.float32),
    )(f, fs, ps, pd, d2s, Wq, Wdot, Wk1, ak, Wv1, av, r16, g16, r32, g32)


def kernel(f, pos, batch, Wq, Wk1, Wk2, Wv1, Wv2, Wdot):
    src, d2s = _build_edges(pos)
    posp = jnp.pad(pos, ((0, 0), (0, 13)))
    dst_flat = (jnp.arange(_E, dtype=jnp.int32) // _K).astype(jnp.int32)
    fs, ps, pd = _sc_gather(f, posp, src.reshape(_E), dst_flat)
    return _core(f, fs, ps, pd, d2s, Wq, Wdot, Wk1, Wk2, Wv1, Wv2)


# speedup vs baseline: 2.8778x; 1.0533x over previous
import functools

import jax
import jax.numpy as jnp
from jax import lax
from jax.experimental import pallas as pl
from jax.experimental.pallas import tpu as pltpu
from jax.experimental.pallas import tpu_sc as plsc

_N = 10000
_DIN = 32
_DOUT = 32
_DK = 16
_DQ = 16
_NB = 10
_HID = 16
_R = 0.22
_K = 16

_B1 = 200
_B3 = 200
_EB = _B3 * _K
_E = _N * _K
_TW = 48

_NW = 32
_PERW = _E // _NW
_CH = 1000


def _edges_body(pos_ref, post_ref, src_ref, d2_ref):
    pb = pos_ref[...]
    pt = post_ref[...]
    x2 = jnp.sum(pt * pt, axis=0, keepdims=True)
    p2 = jnp.sum(pb * pb, axis=1, keepdims=True)
    cross = jnp.dot(pb, pt, preferred_element_type=jnp.float32)
    d2 = jnp.maximum(p2 + x2 - 2.0 * cross, 0.0)
    base = pl.program_id(0) * _B1
    col = lax.broadcasted_iota(jnp.int32, (_B1, _N), 1)
    row = lax.broadcasted_iota(jnp.int32, (_B1, _N), 0) + base
    ok = (col != row) & (d2 < _R * _R)
    big = jnp.float32(1e30)
    bign = jnp.float32(float(_N))
    colf = col.astype(jnp.float32)
    score = jnp.where(ok, d2, big)
    srcs = jnp.zeros((_B1, _K), jnp.float32)
    d2s = jnp.full((_B1, _K), big, jnp.float32)
    kl = lax.broadcasted_iota(jnp.int32, (_B1, _K), 1)
    m = jnp.min(score, axis=1, keepdims=True)
    for k in range(_K):
        idxf = jnp.min(jnp.where(score == m, colf, bign),
                       axis=1, keepdims=True)
        srcs = jnp.where(kl == k, idxf, srcs)
        d2s = jnp.where(kl == k, m, d2s)
        score = jnp.where(colf == idxf, big, score)
        m = jnp.min(score, axis=1, keepdims=True)
    src_ref[...] = srcs.astype(jnp.int32)
    d2_ref[...] = d2s


def _build_edges(pos):
    pos_t = pos.T
    return pl.pallas_call(
        _edges_body,
        grid=(_N // _B1,),
        in_specs=[
            pl.BlockSpec((_B1, 3), lambda i: (i, 0)),
            pl.BlockSpec((3, _N), lambda i: (0, 0)),
        ],
        out_specs=[
            pl.BlockSpec((_B1, _K), lambda i: (i, 0)),
            pl.BlockSpec((_B1, _K), lambda i: (i, 0)),
        ],
        out_shape=[
            jax.ShapeDtypeStruct((_N, _K), jnp.int32),
            jax.ShapeDtypeStruct((_N, _K), jnp.float32),
        ],
    )(pos, pos_t)


def _sc_gather(table, idx_flat):
    mesh = plsc.VectorSubcoreMesh(core_axis_name="c", subcore_axis_name="s")

    @functools.partial(
        pl.kernel,
        mesh=mesh,
        compiler_params=pltpu.CompilerParams(use_tc_tiling_on_sc=False),
        out_type=jax.ShapeDtypeStruct((_E, _TW), jnp.float32),
        scratch_types=[
            pltpu.VMEM((_CH,), jnp.int32),
            pltpu.VMEM((_CH, _TW), jnp.float32),
            pltpu.SemaphoreType.DMA,
        ],
    )
    def k(t_hbm, idx_hbm, out_hbm, idx_v, rows_v, sem):
        wid = lax.axis_index("s") * 2 + lax.axis_index("c")
        base = wid * _PERW
        for c in range(_PERW // _CH):
            off = base + c * _CH
            pltpu.sync_copy(idx_hbm.at[pl.ds(off, _CH)], idx_v)
            pltpu.async_copy(t_hbm.at[idx_v], rows_v, sem).wait()
            pltpu.sync_copy(rows_v, out_hbm.at[pl.ds(off, _CH)])

    return k(table, idx_flat)


def _core_body(f_ref, pos_ref, gat_ref, d2e_ref, wq_ref, wdot_ref,
               wk1_ref, ak_ref, wv1_ref, av_ref, o_ref):
    fb = f_ref[...]
    fs = gat_ref[:, :_DIN]
    ps = gat_ref[:, _DIN:_DIN + 3]
    d2e = d2e_ref[...]
    valid_e = d2e < _R * _R
    validf_e = valid_e.astype(jnp.float32)

    dot = lambda a, b: jnp.dot(a, b, preferred_element_type=jnp.float32,
                               precision=jax.lax.Precision.HIGHEST)
    rep = lambda x: jnp.reshape(
        jnp.broadcast_to(x[:, None, :], (_B3, _K, x.shape[1])),
        (_EB, x.shape[1]))

    q = dot(fb, wq_ref[...]) * (1.0 / jnp.sqrt(jnp.float32(_DIN)))
    qw_e = rep(dot(q, wdot_ref[...]))

    dv = ps - rep(pos_ref[...])
    elen_e = jnp.sqrt(jnp.sum(dv * dv, axis=1, keepdims=True) + 1e-12)
    step = _R / (_NB + 1)
    centers = (lax.broadcasted_iota(jnp.int32, (1, _NB), 1).astype(jnp.float32)
               + 1.0) * step
    diff = (elen_e - centers) * (1.0 / step)
    emb = (jnp.exp(-diff * diff) * (1.0 / 1.12)) * (_NB ** 0.5)

    inv_nb = 1.0 / jnp.sqrt(jnp.float32(_NB))
    sqrt2 = jnp.sqrt(jnp.float32(2.0))
    hk = sqrt2 * jax.nn.relu(dot(emb, wk1_ref[...]) * inv_nb)
    hv = sqrt2 * jax.nn.relu(dot(emb, wv1_ref[...]) * inv_nb)

    tp_scale = 1.0 / (jnp.sqrt(jnp.float32(_HID)) * jnp.sqrt(jnp.float32(_DIN)))
    tk3 = jnp.reshape(dot(fs, ak_ref[...]), (_EB, _HID, _DK))
    km = jnp.sum(tk3 * jnp.broadcast_to(hk[:, :, None], (_EB, _HID, _DK)),
                 axis=1) * tp_scale
    tv3 = jnp.reshape(dot(fs, av_ref[...]), (_EB, _HID, _DOUT))
    vm = jnp.sum(tv3 * jnp.broadcast_to(hv[:, :, None], (_EB, _HID, _DOUT)),
                 axis=1) * tp_scale

    lg_e = jnp.sum(qw_e * km, axis=1, keepdims=True) * (
        1.0 / jnp.sqrt(jnp.float32(_DQ * _DK)))
    lg = jnp.reshape(lg_e, (_B3, _K))
    valid = jnp.reshape(validf_e, (_B3, _K)) > 0.0
    lgm = jnp.where(valid, lg, -1e30)
    mx = jnp.max(lgm, axis=1, keepdims=True)
    mx_e = rep(mx)
    ex_e = jnp.where(valid_e, jnp.exp(lg_e - mx_e), 0.0)
    den = jnp.sum(jnp.reshape(ex_e, (_B3, _K)), axis=1, keepdims=True)
    den_e = rep(den)
    alpha_e = ex_e / (den_e + 1e-16)

    xcut = 10.0 * (1.0 - elen_e * (1.0 / _R))
    xp = jnp.where(xcut > 0.0, xcut, 1.0)
    cut_e = jnp.where(xcut > 0.0, jnp.exp(-1.0 / xp), 0.0)

    coef_e = jnp.sqrt(alpha_e * cut_e + 1e-12) * validf_e
    o_ref[...] = jnp.sum(jnp.reshape(coef_e * vm, (_B3, _K, _DOUT)), axis=1)


def _core(f, pos, gat, d2e, Wq, Wdot, Wk1, Wk2, Wv1, Wv2):
    ak = Wk2.reshape(_HID, _DIN, _DK).transpose(1, 0, 2).reshape(_DIN, _HID * _DK)
    av = Wv2.reshape(_HID, _DIN, _DOUT).transpose(1, 0, 2).reshape(_DIN, _HID * _DOUT)
    full = lambda shape: pl.BlockSpec(shape, lambda i: tuple(0 for _ in shape))
    return pl.pallas_call(
        _core_body,
        grid=(_N // _B3,),
        in_specs=[
            pl.BlockSpec((_B3, _DIN), lambda i: (i, 0)),
            pl.BlockSpec((_B3, 3), lambda i: (i, 0)),
            pl.BlockSpec((_EB, _TW), lambda i: (i, 0)),
            pl.BlockSpec((_EB, 1), lambda i: (i, 0)),
            full((_DIN, _DQ)),
            full((_DQ, _DK)),
            full((_NB, _HID)),
            full((_DIN, _HID * _DK)),
            full((_NB, _HID)),
            full((_DIN, _HID * _DOUT)),
        ],
        out_specs=pl.BlockSpec((_B3, _DOUT), lambda i: (i, 0)),
        out_shape=jax.ShapeDtypeStruct((_N, _DOUT), jnp.float32),
    )(f, pos, gat, d2e, Wq, Wdot, Wk1, ak, Wv1, av)


def kernel(f, pos, batch, Wq, Wk1, Wk2, Wv1, Wv2, Wdot):
    src, d2s = _build_edges(pos)
    table = jnp.concatenate(
        [f, pos, jnp.zeros((_N, _TW - _DIN - 3), jnp.float32)], axis=1)
    gat = _sc_gather(table, src.reshape(_E))
    return _core(f, pos, gat, d2s.reshape(_E, 1), Wq, Wdot, Wk1, Wk2, Wv1, Wv2)


# speedup vs baseline: 2.8810x; 1.0011x over previous
import functools

import jax
import jax.numpy as jnp
from jax import lax
from jax.experimental import pallas as pl
from jax.experimental.pallas import tpu as pltpu
from jax.experimental.pallas import tpu_sc as plsc

_N = 10000
_DIN = 32
_DOUT = 32
_DK = 16
_DQ = 16
_NB = 10
_HID = 16
_R = 0.22
_K = 16

_B1 = 200
_B3 = 200
_EB = _B3 * _K
_E = _N * _K
_TW = 48

_NW = 32
_PERW = _E // _NW
_CH = 1000


def _edges_body(pos_ref, post_ref, src_ref, d2_ref):
    pb = pos_ref[...]
    pt = post_ref[...]
    x2 = jnp.sum(pt * pt, axis=0, keepdims=True)
    p2 = jnp.sum(pb * pb, axis=1, keepdims=True)
    cross = jnp.dot(pb, pt, preferred_element_type=jnp.float32)
    d2 = jnp.maximum(p2 + x2 - 2.0 * cross, 0.0)
    base = pl.program_id(0) * _B1
    col = lax.broadcasted_iota(jnp.int32, (_B1, _N), 1)
    row = lax.broadcasted_iota(jnp.int32, (_B1, _N), 0) + base
    ok = (col != row) & (d2 < _R * _R)
    big = jnp.float32(1e30)
    bign = jnp.float32(float(_N))
    colf = col.astype(jnp.float32)
    score = jnp.where(ok, d2, big)
    srcs = jnp.zeros((_B1, _K), jnp.float32)
    d2s = jnp.full((_B1, _K), big, jnp.float32)
    kl = lax.broadcasted_iota(jnp.int32, (_B1, _K), 1)
    m = jnp.min(score, axis=1, keepdims=True)
    for k in range(_K):
        idxf = jnp.min(jnp.where(score == m, colf, bign),
                       axis=1, keepdims=True)
        srcs = jnp.where(kl == k, idxf, srcs)
        d2s = jnp.where(kl == k, m, d2s)
        score = jnp.where(colf == idxf, big, score)
        m = jnp.min(score, axis=1, keepdims=True)
    src_ref[...] = srcs.astype(jnp.int32)
    d2_ref[...] = d2s


def _build_edges(pos):
    pos_t = pos.T
    return pl.pallas_call(
        _edges_body,
        grid=(_N // _B1,),
        in_specs=[
            pl.BlockSpec((_B1, 3), lambda i: (i, 0)),
            pl.BlockSpec((3, _N), lambda i: (0, 0)),
        ],
        out_specs=[
            pl.BlockSpec((_B1, _K), lambda i: (i, 0)),
            pl.BlockSpec((_B1, _K), lambda i: (i, 0)),
        ],
        out_shape=[
            jax.ShapeDtypeStruct((_N, _K), jnp.int32),
            jax.ShapeDtypeStruct((_N, _K), jnp.float32),
        ],
    )(pos, pos_t)


def _sc_gather(table, idx_flat):
    mesh = plsc.VectorSubcoreMesh(core_axis_name="c", subcore_axis_name="s")

    nch = _PERW // _CH

    @functools.partial(
        pl.kernel,
        mesh=mesh,
        compiler_params=pltpu.CompilerParams(use_tc_tiling_on_sc=False),
        out_type=jax.ShapeDtypeStruct((_E, _TW), jnp.float32),
        scratch_types=[
            pltpu.VMEM((_PERW,), jnp.int32),
            pltpu.VMEM((_CH, _TW), jnp.float32),
            pltpu.VMEM((_CH, _TW), jnp.float32),
            pltpu.SemaphoreType.DMA,
            pltpu.SemaphoreType.DMA,
            pltpu.SemaphoreType.DMA,
            pltpu.SemaphoreType.DMA,
        ],
    )
    def k(t_hbm, idx_hbm, out_hbm, idx_v, rows0, rows1, sg0, sg1, sw0, sw1):
        wid = lax.axis_index("s") * 2 + lax.axis_index("c")
        base = wid * _PERW
        pltpu.sync_copy(idx_hbm.at[pl.ds(base, _PERW)], idx_v)
        rows = (rows0, rows1)
        sg = (sg0, sg1)
        sw = (sw0, sw1)
        g = [None, None]
        w = [None, None]

        def gather(c):
            b = c % 2
            g[b] = pltpu.async_copy(
                t_hbm.at[idx_v.at[pl.ds(c * _CH, _CH)]], rows[b], sg[b])

        gather(0)
        for c in range(nch):
            b = c % 2
            if c + 1 < nch:
                nb = (c + 1) % 2
                if w[nb] is not None:
                    w[nb].wait()
                gather(c + 1)
            g[b].wait()
            w[b] = pltpu.async_copy(
                rows[b], out_hbm.at[pl.ds(base + c * _CH, _CH)], sw[b])
        for b in range(2):
            if w[b] is not None:
                w[b].wait()

    return k(table, idx_flat)


def _core_body(f_ref, pos_ref, gat_ref, d2e_ref, wq_ref, wdot_ref,
               wk1_ref, ak_ref, wv1_ref, av_ref, o_ref):
    fb = f_ref[...]
    fs = gat_ref[:, :_DIN]
    ps = gat_ref[:, _DIN:_DIN + 3]
    d2e = d2e_ref[...]
    valid_e = d2e < _R * _R
    validf_e = valid_e.astype(jnp.float32)

    dot = lambda a, b: jnp.dot(a, b, preferred_element_type=jnp.float32,
                               precision=jax.lax.Precision.HIGHEST)
    rep = lambda x: jnp.reshape(
        jnp.broadcast_to(x[:, None, :], (_B3, _K, x.shape[1])),
        (_EB, x.shape[1]))

    q = dot(fb, wq_ref[...]) * (1.0 / jnp.sqrt(jnp.float32(_DIN)))
    qw_e = rep(dot(q, wdot_ref[...]))

    dv = ps - rep(pos_ref[...])
    elen_e = jnp.sqrt(jnp.sum(dv * dv, axis=1, keepdims=True) + 1e-12)
    step = _R / (_NB + 1)
    centers = (lax.broadcasted_iota(jnp.int32, (1, _NB), 1).astype(jnp.float32)
               + 1.0) * step
    diff = (elen_e - centers) * (1.0 / step)
    emb = (jnp.exp(-diff * diff) * (1.0 / 1.12)) * (_NB ** 0.5)

    inv_nb = 1.0 / jnp.sqrt(jnp.float32(_NB))
    sqrt2 = jnp.sqrt(jnp.float32(2.0))
    hk = sqrt2 * jax.nn.relu(dot(emb, wk1_ref[...]) * inv_nb)
    hv = sqrt2 * jax.nn.relu(dot(emb, wv1_ref[...]) * inv_nb)

    tp_scale = 1.0 / (jnp.sqrt(jnp.float32(_HID)) * jnp.sqrt(jnp.float32(_DIN)))
    tk3 = jnp.reshape(dot(fs, ak_ref[...]), (_EB, _HID, _DK))
    km = jnp.sum(tk3 * jnp.broadcast_to(hk[:, :, None], (_EB, _HID, _DK)),
                 axis=1) * tp_scale
    tv3 = jnp.reshape(dot(fs, av_ref[...]), (_EB, _HID, _DOUT))
    vm = jnp.sum(tv3 * jnp.broadcast_to(hv[:, :, None], (_EB, _HID, _DOUT)),
                 axis=1) * tp_scale

    lg_e = jnp.sum(qw_e * km, axis=1, keepdims=True) * (
        1.0 / jnp.sqrt(jnp.float32(_DQ * _DK)))
    lg = jnp.reshape(lg_e, (_B3, _K))
    valid = jnp.reshape(validf_e, (_B3, _K)) > 0.0
    lgm = jnp.where(valid, lg, -1e30)
    mx = jnp.max(lgm, axis=1, keepdims=True)
    mx_e = rep(mx)
    ex_e = jnp.where(valid_e, jnp.exp(lg_e - mx_e), 0.0)
    den = jnp.sum(jnp.reshape(ex_e, (_B3, _K)), axis=1, keepdims=True)
    den_e = rep(den)
    alpha_e = ex_e / (den_e + 1e-16)

    xcut = 10.0 * (1.0 - elen_e * (1.0 / _R))
    xp = jnp.where(xcut > 0.0, xcut, 1.0)
    cut_e = jnp.where(xcut > 0.0, jnp.exp(-1.0 / xp), 0.0)

    coef_e = jnp.sqrt(alpha_e * cut_e + 1e-12) * validf_e
    o_ref[...] = jnp.sum(jnp.reshape(coef_e * vm, (_B3, _K, _DOUT)), axis=1)


def _core(f, pos, gat, d2e, Wq, Wdot, Wk1, Wk2, Wv1, Wv2):
    ak = Wk2.reshape(_HID, _DIN, _DK).transpose(1, 0, 2).reshape(_DIN, _HID * _DK)
    av = Wv2.reshape(_HID, _DIN, _DOUT).transpose(1, 0, 2).reshape(_DIN, _HID * _DOUT)
    full = lambda shape: pl.BlockSpec(shape, lambda i: tuple(0 for _ in shape))
    return pl.pallas_call(
        _core_body,
        grid=(_N // _B3,),
        in_specs=[
            pl.BlockSpec((_B3, _DIN), lambda i: (i, 0)),
            pl.BlockSpec((_B3, 3), lambda i: (i, 0)),
            pl.BlockSpec((_EB, _TW), lambda i: (i, 0)),
            pl.BlockSpec((_EB, 1), lambda i: (i, 0)),
            full((_DIN, _DQ)),
            full((_DQ, _DK)),
            full((_NB, _HID)),
            full((_DIN, _HID * _DK)),
            full((_NB, _HID)),
            full((_DIN, _HID * _DOUT)),
        ],
        out_specs=pl.BlockSpec((_B3, _DOUT), lambda i: (i, 0)),
        out_shape=jax.ShapeDtypeStruct((_N, _DOUT), jnp.float32),
    )(f, pos, gat, d2e, Wq, Wdot, Wk1, ak, Wv1, av)


def kernel(f, pos, batch, Wq, Wk1, Wk2, Wv1, Wv2, Wdot):
    src, d2s = _build_edges(pos)
    table = jnp.concatenate(
        [f, pos, jnp.zeros((_N, _TW - _DIN - 3), jnp.float32)], axis=1)
    gat = _sc_gather(table, src.reshape(_E))
    return _core(f, pos, gat, d2s.reshape(_E, 1), Wq, Wdot, Wk1, Wk2, Wv1, Wv2)
